# baseline (speedup 1.0000x reference)
import jax
import jax.numpy as jnp
from jax import lax
from jax.experimental import pallas as pl
from jax.experimental.pallas import tpu as pltpu
from jax.experimental.pallas import tpu_sc as plsc

NC = 2
NS = 16
L = 16
NW = NC * NS
W = 128

B = 16384
D = 32
N_EX = 100000
N_EX_PAD = 782 * 128
BPW = B // NW
BCH = 128
NBCH = BPW // BCH
NBUF = 2


def _mirt_body(user_hbm, item_hbm, theta_hbm, a_hbm, b_hbm, out_hbm, *scr):
    u_all, it_all = scr[0], scr[1]
    idx_t = scr[2]
    idx_a = scr[3]
    idx_b = scr[4]
    th_v = scr[5]
    a_v = scr[6]
    b_v = scr[7]
    out_v = scr[8]
    sem_t = scr[9:9 + NBCH]
    sem_a = scr[9 + NBCH:9 + 2 * NBCH]
    sem_b = scr[9 + 2 * NBCH:9 + 3 * NBCH]

    wid = lax.axis_index("s") * NC + lax.axis_index("c")
    base = wid * BPW

    pltpu.sync_copy(user_hbm.at[pl.ds(base, BPW)], u_all)
    pltpu.sync_copy(item_hbm.at[pl.ds(base, BPW)], it_all)

    for k in range(BPW // L):
        c, hi = divmod(k, BCH // L)
        s = pl.ds(k * L, L)
        d = pl.ds(hi * L, L)
        idx_t[c, d] = u_all[s] >> 2
        idx_a[c, d] = it_all[s] >> 2
        idx_b[c, d] = it_all[s] >> 7

    def fire(c):
        dst = pl.ds((c % NBUF) * BCH, BCH)
        return (
            pltpu.async_copy(theta_hbm.at[idx_t.at[c]], th_v.at[dst, :],
                             sem_t[c]),
            pltpu.async_copy(a_hbm.at[idx_a.at[c]], a_v.at[dst, :],
                             sem_a[c]),
            pltpu.async_copy(b_hbm.at[idx_b.at[c]], b_v.at[dst, :],
                             sem_b[c]),
        )

    copies = {c: fire(c) for c in range(min(NBUF, NBCH))}

    lane = lax.iota(jnp.int32, L)

    for c in range(NBCH):
        for cp in copies.pop(c):
            cp.wait()
        rbase = (c % NBUF) * BCH

        def group(hi, _, c=c, rbase=rbase):
            r = c * BCH + hi * L
            s = pl.ds(r, L)
            u = u_all[s]
            it = it_all[s]
            rows = rbase + hi * L + lane
            toff = (u & 3) * D
            aoff = (it & 3) * D
            acc = jnp.zeros((L,), jnp.float32)
            for j in range(D):
                t = plsc.load_gather(th_v, [rows, toff + j])
                av = plsc.load_gather(a_v, [rows, aoff + j])
                acc = acc + (t + t) / (1.0 + jnp.exp(-av))
            bv = plsc.load_gather(b_v, [rows, it & (W - 1)])
            out_v[s] = 1.0 / (1.0 + jnp.exp(bv - acc))
            return 0

        lax.fori_loop(0, BCH // L, group, 0)

        if c + NBUF < NBCH:
            copies[c + NBUF] = fire(c + NBUF)

    pltpu.sync_copy(out_v, out_hbm.at[pl.ds(base, BPW)])


@jax.jit
def kernel(user, item, theta_table, a_table, b_table):
    theta_v = theta_table.reshape(-1, W)
    a_v = a_table.reshape(-1, W)
    b_flat = b_table.reshape(-1)
    b_blocks = jnp.pad(b_flat, (0, N_EX_PAD - N_EX)).reshape(-1, W)
    mesh = plsc.VectorSubcoreMesh(
        core_axis_name="c", subcore_axis_name="s",
        num_cores=NC, num_subcores=NS)
    scratch = (
        [pltpu.VMEM((BPW,), jnp.int32) for _ in range(2)]
        + [pltpu.VMEM((NBCH, BCH), jnp.int32) for _ in range(3)]
        + [pltpu.VMEM((NBUF * BCH, W), jnp.float32) for _ in range(3)]
        + [pltpu.VMEM((BPW,), jnp.float32)]
        + [pltpu.SemaphoreType.DMA for _ in range(3 * NBCH)]
    )
    f = pl.kernel(
        _mirt_body,
        out_type=jax.ShapeDtypeStruct((B,), jnp.float32),
        mesh=mesh,
        compiler_params=pltpu.CompilerParams(needs_layout_passes=False),
        scratch_types=scratch,
    )
    return f(user, item, theta_v, a_v, b_blocks)


# speedup vs baseline: 1.4660x; 1.4660x over previous
import jax
import jax.numpy as jnp
from jax import lax
from jax.experimental import pallas as pl
from jax.experimental.pallas import tpu as pltpu
from jax.experimental.pallas import tpu_sc as plsc

NC = 2
NS = 16
L = 16
NW = NC * NS

B = 16384
D = 32
N_EX = 100000
N_EX_PAD = 782 * 128
BPW = B // NW
NG = BPW // L
SLOT_G = 4
NSLOT = SLOT_G * L
BCH = 128
NBCH = BPW // BCH


def _mirt_body(user_hbm, item_hbm, theta_hbm, a_hbm, b_hbm, out_hbm, *scr):
    u_all, it_all = scr[0], scr[1]
    idx_ib = scr[2]
    th_ring = scr[3]
    a_ring = scr[4]
    b_v = scr[5]
    out_v = scr[6]
    sem_t = scr[7:7 + SLOT_G]
    sem_a = scr[7 + SLOT_G:7 + 2 * SLOT_G]
    sem_b = scr[7 + 2 * SLOT_G]

    wid = lax.axis_index("s") * NC + lax.axis_index("c")
    base = wid * BPW

    pltpu.sync_copy(user_hbm.at[pl.ds(base, BPW)], u_all)
    pltpu.sync_copy(item_hbm.at[pl.ds(base, BPW)], it_all)

    for k in range(BPW // L):
        c, hi = divmod(k, BCH // L)
        idx_ib[c, pl.ds(hi * L, L)] = it_all[pl.ds(k * L, L)] >> 7
    b_copies = [
        pltpu.async_copy(b_hbm.at[idx_ib.at[c]],
                         b_v.at[pl.ds(c * BCH, BCH), :], sem_b)
        for c in range(NBCH)
    ]

    prime = []
    for g in range(SLOT_G):
        uvec = u_all[pl.ds(g * L, L)]
        itvec = it_all[pl.ds(g * L, L)]
        for i in range(L):
            row = g * L + i
            ct = pltpu.async_copy(theta_hbm.at[uvec[i]], th_ring.at[row],
                                  sem_t[g])
            ca = pltpu.async_copy(a_hbm.at[itvec[i]], a_ring.at[row],
                                  sem_a[g])
            prime.append((ct, ca))

    for bc in b_copies:
        bc.wait()

    lane = lax.iota(jnp.int32, L)

    def super_body(p, _):
        for gi in range(SLOT_G):
            g = p * SLOT_G + gi
            for i in range(L):
                prime[gi * L + i][0].wait()
                prime[gi * L + i][1].wait()

            slots = gi * L + lane
            s = pl.ds(g * L, L)
            it = it_all[s]
            acc = jnp.zeros((L,), jnp.float32)
            for j in range(D):
                jj = jnp.full((L,), j, jnp.int32)
                t = plsc.load_gather(th_ring, [slots, jj])
                av = plsc.load_gather(a_ring, [slots, jj])
                acc = acc + (t + t) / (1.0 + jnp.exp(-av))
            bv = plsc.load_gather(b_v, [g * L + lane, it & 127])
            res = 1.0 / (1.0 + jnp.exp(bv - acc))
            out_v[pl.ds(g * L, L)] = res

            @pl.when(g + SLOT_G < NG)
            def _(g=g, gi=gi):
                gn = g + SLOT_G
                uvec = u_all[pl.ds(gn * L, L)]
                itvec = it_all[pl.ds(gn * L, L)]
                for i in range(L):
                    slot = gi * L + i
                    pltpu.async_copy(theta_hbm.at[uvec[i]],
                                     th_ring.at[slot], sem_t[gi])
                    pltpu.async_copy(a_hbm.at[itvec[i]],
                                     a_ring.at[slot], sem_a[gi])

        return 0

    lax.fori_loop(0, NG // SLOT_G, super_body, 0)

    pltpu.sync_copy(out_v, out_hbm.at[pl.ds(base, BPW)])


@jax.jit
def kernel(user, item, theta_table, a_table, b_table):
    b_flat = b_table.reshape(-1)
    b_blocks = jnp.pad(b_flat, (0, N_EX_PAD - N_EX)).reshape(-1, 128)
    mesh = plsc.VectorSubcoreMesh(
        core_axis_name="c", subcore_axis_name="s",
        num_cores=NC, num_subcores=NS)
    scratch = (
        [pltpu.VMEM((BPW,), jnp.int32) for _ in range(2)]
        + [pltpu.VMEM((NBCH, BCH), jnp.int32)]
        + [pltpu.VMEM((NSLOT, D), jnp.float32) for _ in range(2)]
        + [pltpu.VMEM((BPW, 128), jnp.float32)]
        + [pltpu.VMEM((BPW,), jnp.float32)]
        + [pltpu.SemaphoreType.DMA for _ in range(2 * SLOT_G + 1)]
    )
    f = pl.kernel(
        _mirt_body,
        out_type=jax.ShapeDtypeStruct((B,), jnp.float32),
        mesh=mesh,
        compiler_params=pltpu.CompilerParams(
            needs_layout_passes=False, use_tc_tiling_on_sc=True),
        scratch_types=scratch,
    )
    return f(user, item, theta_table, a_table, b_blocks)
